# baseline (speedup 1.0000x reference)
import jax
import jax.numpy as jnp
from jax.experimental import pallas as pl
from jax.experimental.pallas import tpu as pltpu

_K = 512


def _fused(cls_ref, tgt_ref, lp_ref, lt_ref, sc_ref, sl_ref, ce_s, ll_s, ls_s):
    i = pl.program_id(0)
    nc = cls_ref.shape[0]
    lp = cls_ref[...]
    m = jnp.max(lp, axis=0, keepdims=True)
    s = jnp.sum(jnp.exp(lp - m), axis=0, keepdims=True)
    lse = m + jnp.log(s)
    tgt = tgt_ref[0]
    row = jax.lax.broadcasted_iota(jnp.int32, lp.shape, 0)
    idxc = jnp.clip(tgt, 0, nc - 1)
    logit_t = jnp.sum(jnp.where(row == idxc, lp, 0.0), axis=0, keepdims=True)
    ce = jnp.where(tgt != -1, lse - logit_t, 0.0)
    d = jnp.abs(lp_ref[...] - lt_ref[...])
    sl1 = jnp.where(d < 1.0, 0.5 * d * d, d - 0.5)
    ll = jnp.sum(sl1, axis=0, keepdims=True)
    ce_s[pl.ds(i, 1), :] = ce
    ll_s[pl.ds(i, 1), :] = ll
    ls_s[pl.ds(i, 1), :] = ce + ll

    @pl.when(i == pl.num_programs(0) - 1)
    def _():
        g, br = ls_s.shape
        bits = jax.lax.bitcast_convert_type(ls_s[...], jnp.int32)
        v = jnp.int32(0)
        for b in range(30, -1, -1):
            cand = v | jnp.int32(1 << b)
            cnt = jnp.sum((bits >= cand).astype(jnp.int32))
            v = jnp.where(cnt >= _K, cand, v)
        n_gt = jnp.sum((bits > v).astype(jnp.int32))
        need = _K - n_gt
        eq = bits == v
        iv = (jax.lax.broadcasted_iota(jnp.int32, (g, br), 0) * br
              + jax.lax.broadcasted_iota(jnp.int32, (g, br), 1))
        jm = jnp.int32(0)
        for b in range(14, -1, -1):
            cand = jm | jnp.int32(1 << b)
            f = jnp.sum((eq & (iv < cand)).astype(jnp.int32))
            jm = jnp.where(f < need, cand, jm)
        sel = ((bits > v) | (eq & (iv <= jm))).astype(jnp.float32)
        sc_ref[...] = jnp.sum(ce_s[...] * sel, keepdims=True)
        sl_ref[...] = jnp.sum(ll_s[...] * sel, keepdims=True)


def kernel(batch_size, cls_pred, cls_target, loc_pred, loc_target):
    r, c = cls_pred.shape
    g = 32
    rp = ((r + (128 * g) - 1) // (128 * g)) * (128 * g)
    br = rp // g
    pad = rp - r
    cpt = jnp.pad(cls_pred.T, ((0, 0), (0, pad)))
    lpt = jnp.pad(loc_pred.T, ((0, 0), (0, pad)))
    ltt = jnp.pad(loc_target.T, ((0, 0), (0, pad)))
    tg3 = jnp.pad(cls_target.astype(jnp.int32), (0, pad),
                  constant_values=-1).reshape(g, 1, br)
    sc, sl = pl.pallas_call(
        _fused,
        grid=(g,),
        in_specs=[
            pl.BlockSpec((c, br), lambda i: (0, i)),
            pl.BlockSpec((1, 1, br), lambda i: (i, 0, 0)),
            pl.BlockSpec((4, br), lambda i: (0, i)),
            pl.BlockSpec((4, br), lambda i: (0, i)),
        ],
        out_specs=[pl.BlockSpec((1, 1), lambda i: (0, 0))] * 2,
        out_shape=[jax.ShapeDtypeStruct((1, 1), jnp.float32)] * 2,
        scratch_shapes=[pltpu.VMEM((g, br), jnp.float32)] * 3,
    )(cpt, tg3, lpt, ltt)
    bs = jnp.asarray(batch_size, jnp.float32)
    return (sc[0, 0] / bs, sl[0, 0] / bs)
